# speedup vs baseline: 6.0557x; 6.0557x over previous
import functools

import jax
import jax.numpy as jnp
from jax import lax
from jax.experimental import pallas as pl
from jax.experimental.pallas import tpu as pltpu
from jax.experimental.pallas import tpu_sc as plsc

_N = 10000
_E = 320000
_D = 128
_NP = 10240
_NC, _NS = 2, 16
_NW = _NC * _NS
_CH = 128
_NCHUNK = _E // _CH
_RPS = _NP // _NS

_mesh = plsc.VectorSubcoreMesh(core_axis_name="c", subcore_axis_name="s")


def _nchunks(w):
    full = _NCHUNK // _NW
    rem = _NCHUNK - full * _NW
    return full + jnp.where(w < rem, 1, 0).astype(jnp.int32)


@functools.partial(
    pl.kernel,
    out_type=(jax.ShapeDtypeStruct((_NC, _NP), jnp.float32),
              jax.ShapeDtypeStruct((_NC, _NP), jnp.float32)),
    mesh=_mesh,
    scratch_types=(
        pltpu.VMEM((_CH,), jnp.int32),
        pltpu.VMEM((_CH,), jnp.int32),
        pltpu.VMEM((_CH,), jnp.float32),
        pltpu.VMEM((_RPS,), jnp.float32),
        pltpu.VMEM_SHARED((_NP,), jnp.float32),
        pltpu.VMEM_SHARED((_NP,), jnp.float32),
    ),
)
def _degrees(src_hbm, dst_hbm, outs_hbm, outd_hbm,
             sidx, didx, ones_v, zer_v, accs, accd):
    cid = lax.axis_index("c")
    sid = lax.axis_index("s")
    w = sid * _NC + cid
    one = jnp.full((16,), 1.0, jnp.float32)
    zero = jnp.zeros((16,), jnp.float32)
    for j in range(_CH // 16):
        ones_v[pl.ds(16 * j, 16)] = one
    for j in range(_RPS // 16):
        zer_v[pl.ds(16 * j, 16)] = zero
    base = sid * _RPS
    pltpu.sync_copy(zer_v, accs.at[pl.ds(base, _RPS)])
    pltpu.sync_copy(zer_v, accd.at[pl.ds(base, _RPS)])
    plsc.subcore_barrier()

    def body(i, carry):
        off = (w + i * _NW) * _CH
        pltpu.sync_copy(src_hbm.at[pl.ds(off, _CH)], sidx)
        pltpu.sync_copy(dst_hbm.at[pl.ds(off, _CH)], didx)
        pltpu.sync_copy(ones_v, accs.at[sidx], add=True)
        pltpu.sync_copy(ones_v, accd.at[didx], add=True)
        return carry

    lax.fori_loop(0, _nchunks(w), body, 0)
    plsc.subcore_barrier()
    pltpu.sync_copy(accs.at[pl.ds(base, _RPS)],
                    outs_hbm.at[cid, pl.ds(base, _RPS)])
    pltpu.sync_copy(accd.at[pl.ds(base, _RPS)],
                    outd_hbm.at[cid, pl.ds(base, _RPS)])


@functools.partial(
    pl.kernel,
    out_type=jax.ShapeDtypeStruct((_NC, _NP, _D), jnp.float32),
    mesh=_mesh,
    scratch_types=(
        pltpu.VMEM((_CH,), jnp.int32),
        pltpu.VMEM((_CH,), jnp.int32),
        pltpu.VMEM((_CH, _D), jnp.float32),
        pltpu.VMEM((64, _D), jnp.float32),
        pltpu.VMEM_SHARED((_NP, _D), jnp.float32),
    ),
)
def _aggregate(h_hbm, src_hbm, dst_hbm, out_hbm, sidx, didx, rows, zrows, acc):
    cid = lax.axis_index("c")
    sid = lax.axis_index("s")
    w = sid * _NC + cid
    zero = jnp.zeros((16,), jnp.float32)

    def zb(r, carry):
        for j in range(_D // 16):
            zrows[r, pl.ds(16 * j, 16)] = zero
        return carry

    lax.fori_loop(0, 64, zb, 0)
    rowbase = sid * _RPS

    def zc(k, carry):
        pltpu.sync_copy(zrows, acc.at[pl.ds(rowbase + 64 * k, 64)])
        return carry

    lax.fori_loop(0, _RPS // 64, zc, 0)
    plsc.subcore_barrier()

    def body(i, carry):
        off = (w + i * _NW) * _CH
        pltpu.sync_copy(src_hbm.at[pl.ds(off, _CH)], sidx)
        pltpu.sync_copy(dst_hbm.at[pl.ds(off, _CH)], didx)
        pltpu.sync_copy(h_hbm.at[sidx], rows)
        pltpu.sync_copy(rows, acc.at[didx], add=True)
        return carry

    lax.fori_loop(0, _nchunks(w), body, 0)
    plsc.subcore_barrier()
    pltpu.sync_copy(acc.at[pl.ds(rowbase, _RPS)],
                    out_hbm.at[cid, pl.ds(rowbase, _RPS)])


_R = 1024
_G = _NP // _R


def _t1_body(x_ref, s0_ref, s1_ref, w_ref, o_ref):
    ns = lax.rsqrt(jnp.maximum(s0_ref[...] + s1_ref[...], 1.0))
    o_ref[...] = jnp.dot(x_ref[...] * ns, w_ref[...],
                         preferred_element_type=jnp.float32)


_t1 = pl.pallas_call(
    _t1_body,
    grid=(_G,),
    in_specs=[
        pl.BlockSpec((_R, _D), lambda i: (i, 0)),
        pl.BlockSpec((_R, 1), lambda i: (i, 0)),
        pl.BlockSpec((_R, 1), lambda i: (i, 0)),
        pl.BlockSpec((_D, _D), lambda i: (0, 0)),
    ],
    out_specs=pl.BlockSpec((_R, _D), lambda i: (i, 0)),
    out_shape=jax.ShapeDtypeStruct((_NP, _D), jnp.float32),
)


def _tmid_body(agg_ref, d0_ref, d1_ref, s0_ref, s1_ref, b_ref, a_ref, w_ref,
               o_ref):
    h = agg_ref[0] + agg_ref[1]
    nd = lax.rsqrt(jnp.maximum(d0_ref[...] + d1_ref[...], 1.0))
    h = h * nd + b_ref[...]
    h = jnp.where(h >= 0, h, a_ref[...] * h)
    ns = lax.rsqrt(jnp.maximum(s0_ref[...] + s1_ref[...], 1.0))
    o_ref[...] = jnp.dot(h * ns, w_ref[...],
                         preferred_element_type=jnp.float32)


_tmid = pl.pallas_call(
    _tmid_body,
    grid=(_G,),
    in_specs=[
        pl.BlockSpec((_NC, _R, _D), lambda i: (0, i, 0)),
        pl.BlockSpec((_R, 1), lambda i: (i, 0)),
        pl.BlockSpec((_R, 1), lambda i: (i, 0)),
        pl.BlockSpec((_R, 1), lambda i: (i, 0)),
        pl.BlockSpec((_R, 1), lambda i: (i, 0)),
        pl.BlockSpec((1, _D), lambda i: (0, 0)),
        pl.BlockSpec((1, _D), lambda i: (0, 0)),
        pl.BlockSpec((_D, _D), lambda i: (0, 0)),
    ],
    out_specs=pl.BlockSpec((_R, _D), lambda i: (i, 0)),
    out_shape=jax.ShapeDtypeStruct((_NP, _D), jnp.float32),
)


def _t4_body(agg_ref, d0_ref, d1_ref, b_ref, o_ref):
    nd = lax.rsqrt(jnp.maximum(d0_ref[...] + d1_ref[...], 1.0))
    o_ref[...] = (agg_ref[0] + agg_ref[1]) * nd + b_ref[...]


_t4 = pl.pallas_call(
    _t4_body,
    grid=(_G,),
    in_specs=[
        pl.BlockSpec((_NC, _R, _D), lambda i: (0, i, 0)),
        pl.BlockSpec((_R, 1), lambda i: (i, 0)),
        pl.BlockSpec((_R, 1), lambda i: (i, 0)),
        pl.BlockSpec((1, _D), lambda i: (0, 0)),
    ],
    out_specs=pl.BlockSpec((_R, _D), lambda i: (i, 0)),
    out_shape=jax.ShapeDtypeStruct((_NP, _D), jnp.float32),
)


def kernel(feat, edge_index, W1, b1, a1, W2, b2, a2, W3, b3):
    src = edge_index[0]
    dst = edge_index[1]
    degS, degD = _degrees(src, dst)
    s0 = degS[0].reshape(_NP, 1)
    s1 = degS[1].reshape(_NP, 1)
    d0 = degD[0].reshape(_NP, 1)
    d1 = degD[1].reshape(_NP, 1)
    xp = jnp.pad(feat, ((0, _NP - _N), (0, 0)))
    b1r, a1r = b1.reshape(1, _D), a1.reshape(1, _D)
    b2r, a2r = b2.reshape(1, _D), a2.reshape(1, _D)
    b3r = b3.reshape(1, _D)

    h = _t1(xp, s0, s1, W1)
    agg = _aggregate(h, src, dst)
    h = _tmid(agg, d0, d1, s0, s1, b1r, a1r, W2)
    agg = _aggregate(h, src, dst)
    h = _tmid(agg, d0, d1, s0, s1, b2r, a2r, W3)
    agg = _aggregate(h, src, dst)
    out = _t4(agg, d0, d1, b3r)
    return out[:_N]


# speedup vs baseline: 10.6816x; 1.7639x over previous
import functools

import jax
import jax.numpy as jnp
from jax import lax
from jax.experimental import pallas as pl
from jax.experimental.pallas import tpu as pltpu
from jax.experimental.pallas import tpu_sc as plsc

_N = 10000
_E = 320000
_D = 128
_NP = 10240
_NC, _NS = 2, 16
_NW = _NC * _NS
_CH = 64
_CPW = 160
_EP = _NW * _CPW * _CH
_NBUF = 2
_RPS = _NP // _NS

_mesh = plsc.VectorSubcoreMesh(core_axis_name="c", subcore_axis_name="s")


@functools.partial(
    pl.kernel,
    out_type=(jax.ShapeDtypeStruct((_NC, _NP), jnp.float32),
              jax.ShapeDtypeStruct((_NC, _NP), jnp.float32)),
    mesh=_mesh,
    scratch_types=(
        pltpu.VMEM((_CPW * _CH,), jnp.int32),
        pltpu.VMEM((_CPW * _CH,), jnp.int32),
        pltpu.VMEM((_CH,), jnp.int32),
        pltpu.VMEM((_CH,), jnp.int32),
        pltpu.VMEM((_CH,), jnp.float32),
        pltpu.VMEM((_RPS,), jnp.float32),
        pltpu.VMEM_SHARED((_NP,), jnp.float32),
        pltpu.VMEM_SHARED((_NP,), jnp.float32),
    ),
)
def _degrees(src_hbm, dst_hbm, outs_hbm, outd_hbm,
             sidx, didx, sidx_b, didx_b, ones_v, zer_v, accs, accd):
    cid = lax.axis_index("c")
    sid = lax.axis_index("s")
    w = sid * _NC + cid
    one = jnp.full((16,), 1.0, jnp.float32)
    zero = jnp.zeros((16,), jnp.float32)
    for j in range(_CH // 16):
        ones_v[pl.ds(16 * j, 16)] = one
    for j in range(_RPS // 16):
        zer_v[pl.ds(16 * j, 16)] = zero
    base = sid * _RPS
    pltpu.sync_copy(zer_v, accs.at[pl.ds(base, _RPS)])
    pltpu.sync_copy(zer_v, accd.at[pl.ds(base, _RPS)])
    pltpu.sync_copy(src_hbm.at[pl.ds(w * _CPW * _CH, _CPW * _CH)], sidx)
    pltpu.sync_copy(dst_hbm.at[pl.ds(w * _CPW * _CH, _CPW * _CH)], didx)
    plsc.subcore_barrier()

    def body(j, carry):
        for v in range(_CH // 16):
            sidx_b[pl.ds(16 * v, 16)] = sidx[pl.ds(j * _CH + 16 * v, 16)]
            didx_b[pl.ds(16 * v, 16)] = didx[pl.ds(j * _CH + 16 * v, 16)]
        pltpu.sync_copy(ones_v, accs.at[sidx_b], add=True)
        pltpu.sync_copy(ones_v, accd.at[didx_b], add=True)
        return carry

    lax.fori_loop(0, _CPW, body, 0)
    plsc.subcore_barrier()
    pltpu.sync_copy(accs.at[pl.ds(base, _RPS)],
                    outs_hbm.at[cid, pl.ds(base, _RPS)])
    pltpu.sync_copy(accd.at[pl.ds(base, _RPS)],
                    outd_hbm.at[cid, pl.ds(base, _RPS)])


@functools.partial(
    pl.kernel,
    out_type=jax.ShapeDtypeStruct((_NC, _NP, _D), jnp.float32),
    mesh=_mesh,
    scratch_types=(
        pltpu.VMEM((_CPW * _CH,), jnp.int32),
        pltpu.VMEM((_CPW * _CH,), jnp.int32),
        tuple(pltpu.VMEM((_CH,), jnp.int32) for _ in range(_NBUF)),
        tuple(pltpu.VMEM((_CH, _D), jnp.float32) for _ in range(_NBUF)),
        tuple(pltpu.SemaphoreType.DMA for _ in range(_NBUF)),
        pltpu.VMEM_SHARED((_NP, _D), jnp.float32),
    ),
)
def _aggregate(h_hbm, src_hbm, dst_hbm, out_hbm,
               sidx, didx, didx_b, rows, sems, acc):
    cid = lax.axis_index("c")
    sid = lax.axis_index("s")
    w = sid * _NC + cid
    zero = jnp.zeros((16,), jnp.float32)

    def zb(r, carry):
        for j in range(_D // 16):
            rows[0][r, pl.ds(16 * j, 16)] = zero
        return carry

    lax.fori_loop(0, _CH, zb, 0)
    rowbase = sid * _RPS

    def zc(k, carry):
        pltpu.sync_copy(rows[0], acc.at[pl.ds(rowbase + _CH * k, _CH)])
        return carry

    lax.fori_loop(0, _RPS // _CH, zc, 0)
    pltpu.sync_copy(src_hbm.at[pl.ds(w * _CPW * _CH, _CPW * _CH)], sidx)
    pltpu.sync_copy(dst_hbm.at[pl.ds(w * _CPW * _CH, _CPW * _CH)], didx)
    plsc.subcore_barrier()

    def start(k, b):
        pltpu.async_copy(h_hbm.at[sidx.at[pl.ds(k * _CH, _CH)]],
                         rows[b], sems[b])

    def finish(k, b):
        for v in range(_CH // 16):
            didx_b[b][pl.ds(16 * v, 16)] = didx[pl.ds(k * _CH + 16 * v, 16)]
        pltpu.make_async_copy(h_hbm.at[sidx.at[pl.ds(k * _CH, _CH)]],
                              rows[b], sems[b]).wait()
        pltpu.sync_copy(rows[b], acc.at[didx_b[b]], add=True)

    for b in range(_NBUF):
        start(b, b)

    def body(j, carry):
        for b in range(_NBUF):
            k = _NBUF * j + b
            finish(k, b)
            start(k + _NBUF, b)
        return carry

    lax.fori_loop(0, _CPW // _NBUF - 1, body, 0)
    for b in range(_NBUF):
        finish(_CPW - _NBUF + b, b)

    plsc.subcore_barrier()
    pltpu.sync_copy(acc.at[pl.ds(rowbase, _RPS)],
                    out_hbm.at[cid, pl.ds(rowbase, _RPS)])


_R = 1024
_G = _NP // _R


def _t1_body(x_ref, s0_ref, s1_ref, w_ref, o_ref):
    ns = lax.rsqrt(jnp.maximum(s0_ref[...] + s1_ref[...], 1.0))
    o_ref[...] = jnp.dot(x_ref[...] * ns, w_ref[...],
                         preferred_element_type=jnp.float32)


_t1 = pl.pallas_call(
    _t1_body,
    grid=(_G,),
    in_specs=[
        pl.BlockSpec((_R, _D), lambda i: (i, 0)),
        pl.BlockSpec((_R, 1), lambda i: (i, 0)),
        pl.BlockSpec((_R, 1), lambda i: (i, 0)),
        pl.BlockSpec((_D, _D), lambda i: (0, 0)),
    ],
    out_specs=pl.BlockSpec((_R, _D), lambda i: (i, 0)),
    out_shape=jax.ShapeDtypeStruct((_NP, _D), jnp.float32),
)


def _tmid_body(agg_ref, d0_ref, d1_ref, s0_ref, s1_ref, b_ref, a_ref, w_ref,
               o_ref):
    h = agg_ref[0] + agg_ref[1]
    nd = lax.rsqrt(jnp.maximum(d0_ref[...] + d1_ref[...], 1.0))
    h = h * nd + b_ref[...]
    h = jnp.where(h >= 0, h, a_ref[...] * h)
    ns = lax.rsqrt(jnp.maximum(s0_ref[...] + s1_ref[...], 1.0))
    o_ref[...] = jnp.dot(h * ns, w_ref[...],
                         preferred_element_type=jnp.float32)


_tmid = pl.pallas_call(
    _tmid_body,
    grid=(_G,),
    in_specs=[
        pl.BlockSpec((_NC, _R, _D), lambda i: (0, i, 0)),
        pl.BlockSpec((_R, 1), lambda i: (i, 0)),
        pl.BlockSpec((_R, 1), lambda i: (i, 0)),
        pl.BlockSpec((_R, 1), lambda i: (i, 0)),
        pl.BlockSpec((_R, 1), lambda i: (i, 0)),
        pl.BlockSpec((1, _D), lambda i: (0, 0)),
        pl.BlockSpec((1, _D), lambda i: (0, 0)),
        pl.BlockSpec((_D, _D), lambda i: (0, 0)),
    ],
    out_specs=pl.BlockSpec((_R, _D), lambda i: (i, 0)),
    out_shape=jax.ShapeDtypeStruct((_NP, _D), jnp.float32),
)


def _t4_body(agg_ref, d0_ref, d1_ref, b_ref, o_ref):
    nd = lax.rsqrt(jnp.maximum(d0_ref[...] + d1_ref[...], 1.0))
    o_ref[...] = (agg_ref[0] + agg_ref[1]) * nd + b_ref[...]


_t4 = pl.pallas_call(
    _t4_body,
    grid=(_G,),
    in_specs=[
        pl.BlockSpec((_NC, _R, _D), lambda i: (0, i, 0)),
        pl.BlockSpec((_R, 1), lambda i: (i, 0)),
        pl.BlockSpec((_R, 1), lambda i: (i, 0)),
        pl.BlockSpec((1, _D), lambda i: (0, 0)),
    ],
    out_specs=pl.BlockSpec((_R, _D), lambda i: (i, 0)),
    out_shape=jax.ShapeDtypeStruct((_NP, _D), jnp.float32),
)


def kernel(feat, edge_index, W1, b1, a1, W2, b2, a2, W3, b3):
    src = edge_index[0]
    dst = edge_index[1]
    pad = _N + (jnp.arange(_EP - _E, dtype=jnp.int32) % (_NP - _N))
    srcp = jnp.concatenate([src, pad])
    dstp = jnp.concatenate([dst, pad])

    degS, degD = _degrees(srcp, dstp)
    s0 = degS[0].reshape(_NP, 1)
    s1 = degS[1].reshape(_NP, 1)
    d0 = degD[0].reshape(_NP, 1)
    d1 = degD[1].reshape(_NP, 1)
    xp = jnp.pad(feat, ((0, _NP - _N), (0, 0)))
    b1r, a1r = b1.reshape(1, _D), a1.reshape(1, _D)
    b2r, a2r = b2.reshape(1, _D), a2.reshape(1, _D)
    b3r = b3.reshape(1, _D)

    h = _t1(xp, s0, s1, W1)
    agg = _aggregate(h, srcp, dstp)
    h = _tmid(agg, d0, d1, s0, s1, b1r, a1r, W2)
    agg = _aggregate(h, srcp, dstp)
    h = _tmid(agg, d0, d1, s0, s1, b2r, a2r, W3)
    agg = _aggregate(h, srcp, dstp)
    out = _t4(agg, d0, d1, b3r)
    return out[:_N]


# speedup vs baseline: 12.7240x; 1.1912x over previous
import functools

import jax
import jax.numpy as jnp
from jax import lax
from jax.experimental import pallas as pl
from jax.experimental.pallas import tpu as pltpu
from jax.experimental.pallas import tpu_sc as plsc

_N = 10000
_E = 320000
_D = 128
_NP = 10240
_NC, _NS = 2, 16
_NW = _NC * _NS
_CH = 128
_CPW = 80
_EP = _NW * _CPW * _CH
_NBUF = 2
_RPS = _NP // _NS

_mesh = plsc.VectorSubcoreMesh(core_axis_name="c", subcore_axis_name="s")


@functools.partial(
    pl.kernel,
    out_type=(jax.ShapeDtypeStruct((_NC, _NP), jnp.float32),
              jax.ShapeDtypeStruct((_NC, _NP), jnp.float32)),
    mesh=_mesh,
    scratch_types=(
        pltpu.VMEM((_CPW * _CH,), jnp.int32),
        pltpu.VMEM((_CPW * _CH,), jnp.int32),
        pltpu.VMEM((_CH,), jnp.int32),
        pltpu.VMEM((_CH,), jnp.int32),
        pltpu.VMEM((_CH,), jnp.float32),
        pltpu.VMEM((_RPS,), jnp.float32),
        pltpu.VMEM_SHARED((_NP,), jnp.float32),
        pltpu.VMEM_SHARED((_NP,), jnp.float32),
    ),
)
def _degrees(src_hbm, dst_hbm, outs_hbm, outd_hbm,
             sidx, didx, sidx_b, didx_b, ones_v, zer_v, accs, accd):
    cid = lax.axis_index("c")
    sid = lax.axis_index("s")
    w = sid * _NC + cid
    one = jnp.full((16,), 1.0, jnp.float32)
    zero = jnp.zeros((16,), jnp.float32)
    for j in range(_CH // 16):
        ones_v[pl.ds(16 * j, 16)] = one
    for j in range(_RPS // 16):
        zer_v[pl.ds(16 * j, 16)] = zero
    base = sid * _RPS
    pltpu.sync_copy(zer_v, accs.at[pl.ds(base, _RPS)])
    pltpu.sync_copy(zer_v, accd.at[pl.ds(base, _RPS)])
    pltpu.sync_copy(src_hbm.at[pl.ds(w * _CPW * _CH, _CPW * _CH)], sidx)
    pltpu.sync_copy(dst_hbm.at[pl.ds(w * _CPW * _CH, _CPW * _CH)], didx)
    plsc.subcore_barrier()

    def body(j, carry):
        for v in range(_CH // 16):
            sidx_b[pl.ds(16 * v, 16)] = sidx[pl.ds(j * _CH + 16 * v, 16)]
            didx_b[pl.ds(16 * v, 16)] = didx[pl.ds(j * _CH + 16 * v, 16)]
        pltpu.sync_copy(ones_v, accs.at[sidx_b], add=True)
        pltpu.sync_copy(ones_v, accd.at[didx_b], add=True)
        return carry

    lax.fori_loop(0, _CPW, body, 0)
    plsc.subcore_barrier()
    pltpu.sync_copy(accs.at[pl.ds(base, _RPS)],
                    outs_hbm.at[cid, pl.ds(base, _RPS)])
    pltpu.sync_copy(accd.at[pl.ds(base, _RPS)],
                    outd_hbm.at[cid, pl.ds(base, _RPS)])


@functools.partial(
    pl.kernel,
    out_type=jax.ShapeDtypeStruct((_NC, _NP, _D), jnp.float32),
    mesh=_mesh,
    scratch_types=(
        pltpu.VMEM((_CPW * _CH,), jnp.int32),
        tuple(pltpu.VMEM((_CH,), jnp.int32) for _ in range(_NBUF)),
        tuple(pltpu.VMEM((_CH, _D), jnp.float32) for _ in range(_NBUF)),
        tuple(pltpu.SemaphoreType.DMA for _ in range(_NBUF)),
        tuple(pltpu.SemaphoreType.DMA for _ in range(_NBUF)),
        pltpu.SemaphoreType.DMA,
        pltpu.VMEM_SHARED((_NP, _D), jnp.float32),
    ),
)
def _aggregate(h_hbm, src_hbm, dst_hbm, out_hbm,
               sidx, didx_b, rows, gsems, isems, lsem, acc):
    cid = lax.axis_index("c")
    sid = lax.axis_index("s")
    w = sid * _NC + cid
    ebase = w * _CPW * _CH
    zero = jnp.zeros((16,), jnp.float32)

    ldesc = pltpu.async_copy(src_hbm.at[pl.ds(ebase, _CPW * _CH)], sidx, lsem)

    def zb(r, carry):
        for j in range(_D // 16):
            rows[0][r, pl.ds(16 * j, 16)] = zero
        return carry

    lax.fori_loop(0, _CH, zb, 0)
    rowbase = sid * _RPS

    def zc(k, carry):
        pltpu.sync_copy(rows[0], acc.at[pl.ds(rowbase + _CH * k, _CH)])
        return carry

    lax.fori_loop(0, _RPS // _CH, zc, 0)
    ldesc.wait()
    plsc.subcore_barrier()

    def start(k, b):
        pltpu.async_copy(dst_hbm.at[pl.ds(ebase + k * _CH, _CH)],
                         didx_b[b], isems[b])
        pltpu.async_copy(h_hbm.at[sidx.at[pl.ds(k * _CH, _CH)]],
                         rows[b], gsems[b])

    def finish(k, b):
        pltpu.make_async_copy(dst_hbm.at[pl.ds(ebase + k * _CH, _CH)],
                              didx_b[b], isems[b]).wait()
        pltpu.make_async_copy(h_hbm.at[sidx.at[pl.ds(k * _CH, _CH)]],
                              rows[b], gsems[b]).wait()
        pltpu.sync_copy(rows[b], acc.at[didx_b[b]], add=True)

    for b in range(_NBUF):
        start(b, b)

    def body(j, carry):
        for b in range(_NBUF):
            k = _NBUF * j + b
            finish(k, b)
            start(k + _NBUF, b)
        return carry

    lax.fori_loop(0, _CPW // _NBUF - 1, body, 0)
    for b in range(_NBUF):
        finish(_CPW - _NBUF + b, b)

    plsc.subcore_barrier()
    pltpu.sync_copy(acc.at[pl.ds(rowbase, _RPS)],
                    out_hbm.at[cid, pl.ds(rowbase, _RPS)])


_R = 1024
_G = _NP // _R


def _t1_body(x_ref, s0_ref, s1_ref, w_ref, o_ref):
    ns = lax.rsqrt(jnp.maximum(s0_ref[...] + s1_ref[...], 1.0))
    o_ref[...] = jnp.dot(x_ref[...] * ns, w_ref[...],
                         preferred_element_type=jnp.float32)


_t1 = pl.pallas_call(
    _t1_body,
    grid=(_G,),
    in_specs=[
        pl.BlockSpec((_R, _D), lambda i: (i, 0)),
        pl.BlockSpec((_R, 1), lambda i: (i, 0)),
        pl.BlockSpec((_R, 1), lambda i: (i, 0)),
        pl.BlockSpec((_D, _D), lambda i: (0, 0)),
    ],
    out_specs=pl.BlockSpec((_R, _D), lambda i: (i, 0)),
    out_shape=jax.ShapeDtypeStruct((_NP, _D), jnp.float32),
)


def _tmid_body(agg_ref, d0_ref, d1_ref, s0_ref, s1_ref, b_ref, a_ref, w_ref,
               o_ref):
    h = agg_ref[0] + agg_ref[1]
    nd = lax.rsqrt(jnp.maximum(d0_ref[...] + d1_ref[...], 1.0))
    h = h * nd + b_ref[...]
    h = jnp.where(h >= 0, h, a_ref[...] * h)
    ns = lax.rsqrt(jnp.maximum(s0_ref[...] + s1_ref[...], 1.0))
    o_ref[...] = jnp.dot(h * ns, w_ref[...],
                         preferred_element_type=jnp.float32)


_tmid = pl.pallas_call(
    _tmid_body,
    grid=(_G,),
    in_specs=[
        pl.BlockSpec((_NC, _R, _D), lambda i: (0, i, 0)),
        pl.BlockSpec((_R, 1), lambda i: (i, 0)),
        pl.BlockSpec((_R, 1), lambda i: (i, 0)),
        pl.BlockSpec((_R, 1), lambda i: (i, 0)),
        pl.BlockSpec((_R, 1), lambda i: (i, 0)),
        pl.BlockSpec((1, _D), lambda i: (0, 0)),
        pl.BlockSpec((1, _D), lambda i: (0, 0)),
        pl.BlockSpec((_D, _D), lambda i: (0, 0)),
    ],
    out_specs=pl.BlockSpec((_R, _D), lambda i: (i, 0)),
    out_shape=jax.ShapeDtypeStruct((_NP, _D), jnp.float32),
)


def _t4_body(agg_ref, d0_ref, d1_ref, b_ref, o_ref):
    nd = lax.rsqrt(jnp.maximum(d0_ref[...] + d1_ref[...], 1.0))
    o_ref[...] = (agg_ref[0] + agg_ref[1]) * nd + b_ref[...]


_t4 = pl.pallas_call(
    _t4_body,
    grid=(_G,),
    in_specs=[
        pl.BlockSpec((_NC, _R, _D), lambda i: (0, i, 0)),
        pl.BlockSpec((_R, 1), lambda i: (i, 0)),
        pl.BlockSpec((_R, 1), lambda i: (i, 0)),
        pl.BlockSpec((1, _D), lambda i: (0, 0)),
    ],
    out_specs=pl.BlockSpec((_R, _D), lambda i: (i, 0)),
    out_shape=jax.ShapeDtypeStruct((_NP, _D), jnp.float32),
)


def kernel(feat, edge_index, W1, b1, a1, W2, b2, a2, W3, b3):
    src = edge_index[0]
    dst = edge_index[1]
    pad = _N + (jnp.arange(_EP - _E, dtype=jnp.int32) % (_NP - _N))
    srcp = jnp.concatenate([src, pad])
    dstp = jnp.concatenate([dst, pad])

    degS, degD = _degrees(srcp, dstp)
    s0 = degS[0].reshape(_NP, 1)
    s1 = degS[1].reshape(_NP, 1)
    d0 = degD[0].reshape(_NP, 1)
    d1 = degD[1].reshape(_NP, 1)
    xp = jnp.pad(feat, ((0, _NP - _N), (0, 0)))
    b1r, a1r = b1.reshape(1, _D), a1.reshape(1, _D)
    b2r, a2r = b2.reshape(1, _D), a2.reshape(1, _D)
    b3r = b3.reshape(1, _D)

    h = _t1(xp, s0, s1, W1)
    agg = _aggregate(h, srcp, dstp)
    h = _tmid(agg, d0, d1, s0, s1, b1r, a1r, W2)
    agg = _aggregate(h, srcp, dstp)
    h = _tmid(agg, d0, d1, s0, s1, b2r, a2r, W3)
    agg = _aggregate(h, srcp, dstp)
    out = _t4(agg, d0, d1, b3r)
    return out[:_N]
